# baseline (speedup 1.0000x reference)
import jax
import jax.numpy as jnp
from jax.experimental import pallas as pl
from jax.experimental.pallas import tpu as pltpu

T = 16384
D = 2048
BT = 1024
BD = 1024


def _add_kernel(idx_ref, emb_ref, x_ref, o_ref):
    i = idx_ref[0]
    row = emb_ref[pl.ds(i, 1), :]
    o_ref[0] = x_ref[...] + row


def kernel(input_features, modality_indices, embedding_weight):
    grid = (T // BT, D // BD)
    out = pl.pallas_call(
        _add_kernel,
        grid_spec=pltpu.PrefetchScalarGridSpec(
            num_scalar_prefetch=1,
            grid=grid,
            in_specs=[
                pl.BlockSpec((4, BD), lambda i, j, idx: (0, j)),
                pl.BlockSpec((BT, BD), lambda i, j, idx: (i, j)),
            ],
            out_specs=pl.BlockSpec((1, BT, BD), lambda i, j, idx: (0, i, j)),
        ),
        out_shape=jax.ShapeDtypeStruct((1, T, D), input_features.dtype),
        compiler_params=pltpu.CompilerParams(
            dimension_semantics=("arbitrary", "arbitrary"),
        ),
    )(modality_indices, embedding_weight, input_features)
    return out


# speedup vs baseline: 1.0082x; 1.0082x over previous
import jax
import jax.numpy as jnp
from jax.experimental import pallas as pl
from jax.experimental.pallas import tpu as pltpu

T = 16384
D = 2048
C = 512
NCH = T // C


def _add_kernel(idx_ref, emb_ref, x_hbm, o_hbm,
                i0, i1, o0, o1, si0, si1, so0, so1):
    ibufs = (i0, i1)
    obufs = (o0, o1)
    sis = (si0, si1)
    sos = (so0, so1)
    idx = idx_ref[0]
    row = emb_ref[pl.ds(idx, 1), :]

    def gather(g, b):
        pltpu.make_async_copy(
            x_hbm.at[pl.ds(g * C, C), :], ibufs[b], sis[b]).start()

    def scatter(g, b):
        pltpu.make_async_copy(
            obufs[b], o_hbm.at[0, pl.ds(g * C, C), :], sos[b]).start()

    gather(0, 0)
    gather(1, 1)
    for g in range(NCH):
        b = g % 2
        pltpu.make_async_copy(
            x_hbm.at[pl.ds(g * C, C), :], ibufs[b], sis[b]).wait()
        if g >= 2:
            pltpu.make_async_copy(
                obufs[b], o_hbm.at[0, pl.ds(0, C), :], sos[b]).wait()
        obufs[b][...] = ibufs[b][...] + row
        scatter(g, b)
        if g + 2 < NCH:
            gather(g + 2, b)
    for b in range(2):
        pltpu.make_async_copy(
            obufs[b], o_hbm.at[0, pl.ds(0, C), :], sos[b]).wait()


def kernel(input_features, modality_indices, embedding_weight):
    out = pl.pallas_call(
        _add_kernel,
        grid_spec=pltpu.PrefetchScalarGridSpec(
            num_scalar_prefetch=1,
            grid=(1,),
            in_specs=[
                pl.BlockSpec((4, D), lambda i, idx: (0, 0)),
                pl.BlockSpec(memory_space=pl.ANY),
            ],
            out_specs=pl.BlockSpec(memory_space=pl.ANY),
            scratch_shapes=[
                pltpu.VMEM((C, D), jnp.float32),
                pltpu.VMEM((C, D), jnp.float32),
                pltpu.VMEM((C, D), jnp.float32),
                pltpu.VMEM((C, D), jnp.float32),
                pltpu.SemaphoreType.DMA,
                pltpu.SemaphoreType.DMA,
                pltpu.SemaphoreType.DMA,
                pltpu.SemaphoreType.DMA,
            ],
        ),
        out_shape=jax.ShapeDtypeStruct((1, T, D), input_features.dtype),
    )(modality_indices, embedding_weight, input_features)
    return out


# speedup vs baseline: 1.0205x; 1.0122x over previous
import jax
import jax.numpy as jnp
from jax.experimental import pallas as pl
from jax.experimental.pallas import tpu as pltpu

T = 16384
D = 2048
C = 1024
NCH = T // C


def _add_kernel(idx_ref, emb_ref, x_hbm, o_hbm,
                i0, i1, o0, o1, si0, si1, so0, so1):
    ibufs = (i0, i1)
    obufs = (o0, o1)
    sis = (si0, si1)
    sos = (so0, so1)
    idx = idx_ref[0]
    row = emb_ref[pl.ds(idx, 1), :]

    def gather(g, b):
        pltpu.make_async_copy(
            x_hbm.at[pl.ds(g * C, C), :], ibufs[b], sis[b]).start()

    def scatter(g, b):
        pltpu.make_async_copy(
            obufs[b], o_hbm.at[0, pl.ds(g * C, C), :], sos[b]).start()

    gather(0, 0)
    gather(1, 1)
    for g in range(NCH):
        b = g % 2
        pltpu.make_async_copy(
            x_hbm.at[pl.ds(g * C, C), :], ibufs[b], sis[b]).wait()
        if g >= 2:
            pltpu.make_async_copy(
                obufs[b], o_hbm.at[0, pl.ds(0, C), :], sos[b]).wait()
        obufs[b][...] = ibufs[b][...] + row
        scatter(g, b)
        if g + 2 < NCH:
            gather(g + 2, b)
    for b in range(2):
        pltpu.make_async_copy(
            obufs[b], o_hbm.at[0, pl.ds(0, C), :], sos[b]).wait()


def kernel(input_features, modality_indices, embedding_weight):
    out = pl.pallas_call(
        _add_kernel,
        grid_spec=pltpu.PrefetchScalarGridSpec(
            num_scalar_prefetch=1,
            grid=(1,),
            in_specs=[
                pl.BlockSpec((4, D), lambda i, idx: (0, 0)),
                pl.BlockSpec(memory_space=pl.ANY),
            ],
            out_specs=pl.BlockSpec(memory_space=pl.ANY),
            scratch_shapes=[
                pltpu.VMEM((C, D), jnp.float32),
                pltpu.VMEM((C, D), jnp.float32),
                pltpu.VMEM((C, D), jnp.float32),
                pltpu.VMEM((C, D), jnp.float32),
                pltpu.SemaphoreType.DMA,
                pltpu.SemaphoreType.DMA,
                pltpu.SemaphoreType.DMA,
                pltpu.SemaphoreType.DMA,
            ],
        ),
        out_shape=jax.ShapeDtypeStruct((1, T, D), input_features.dtype),
    )(modality_indices, embedding_weight, input_features)
    return out


# speedup vs baseline: 1.0282x; 1.0075x over previous
import jax
import jax.numpy as jnp
from jax.experimental import pallas as pl
from jax.experimental.pallas import tpu as pltpu

T = 16384
D = 2048
BT = 1024


def _add_kernel(idx_ref, emb_ref, x_ref, o_ref):
    i = idx_ref[0]
    row = emb_ref[pl.ds(i, 1), :]
    o_ref[0] = x_ref[...] + row


def kernel(input_features, modality_indices, embedding_weight):
    grid = (T // BT,)
    out = pl.pallas_call(
        _add_kernel,
        grid_spec=pltpu.PrefetchScalarGridSpec(
            num_scalar_prefetch=1,
            grid=grid,
            in_specs=[
                pl.BlockSpec((4, D), lambda i, idx: (0, 0)),
                pl.BlockSpec((BT, D), lambda i, idx: (i, 0)),
            ],
            out_specs=pl.BlockSpec((1, BT, D), lambda i, idx: (0, i, 0)),
        ),
        out_shape=jax.ShapeDtypeStruct((1, T, D), input_features.dtype),
        compiler_params=pltpu.CompilerParams(
            dimension_semantics=("parallel",),
        ),
    )(modality_indices, embedding_weight, input_features)
    return out
